# baseline (speedup 1.0000x reference)
import jax
import jax.numpy as jnp
from jax import lax
from jax.experimental import pallas as pl
from jax.experimental.pallas import tpu as pltpu
from jax.experimental.pallas import tpu_sc as plsc

N = 10000
E = 320000
NC = 2
NS = 16
CS = 128
NCHUNK = 160
EPT = CS * NCHUNK
EPAD = NS * EPT
ACC_ROWS = 10112
RPT = ACC_ROWS // NS
NBUF = 4
F32 = jnp.float32

_SC_PARAMS = pltpu.CompilerParams(use_tc_tiling_on_sc=False)
_MESH = plsc.VectorSubcoreMesh(
    core_axis_name="c", subcore_axis_name="s",
    num_cores=NC, num_subcores=NS)


def _zero_fill(zero_hbm, buf, ref, base, total, sem_a, sem_b):
    for r in range(CS // 8):
        pltpu.async_copy(zero_hbm, buf.at[pl.ds(8 * r, 8)], sem_a)
    for r in range(CS // 8):
        pltpu.make_async_copy(
            zero_hbm, buf.at[pl.ds(8 * r, 8)], sem_a).wait()
    nfull = total // CS
    for q in range(nfull):
        pltpu.async_copy(buf, ref.at[pl.ds(base + CS * q, CS)], sem_b)
    rem = total - nfull * CS
    if rem:
        pltpu.async_copy(buf.at[pl.ds(0, rem)],
                         ref.at[pl.ds(base + nfull * CS, rem)], sem_b)
    for q in range(nfull):
        pltpu.make_async_copy(
            buf, ref.at[pl.ds(base + CS * q, CS)], sem_b).wait()
    if rem:
        pltpu.make_async_copy(
            buf.at[pl.ds(0, rem)],
            ref.at[pl.ds(base + nfull * CS, rem)], sem_b).wait()


def _make_segsum(WH, with_counts):
    if with_counts:
        out_type = [jax.ShapeDtypeStruct((NC, ACC_ROWS, WH), F32),
                    jax.ShapeDtypeStruct((ACC_ROWS, 16), F32)]
    else:
        out_type = jax.ShapeDtypeStruct((NC, ACC_ROWS, WH), F32)

    scratch = [
        pltpu.VMEM((NCHUNK, CS), jnp.int32),
        pltpu.VMEM((NCHUNK, CS), jnp.int32),
    ]
    scratch += [pltpu.VMEM((CS, WH), F32) for _ in range(NBUF)]
    scratch += [pltpu.VMEM_SHARED((ACC_ROWS, WH), F32)]
    scratch += [pltpu.SemaphoreType.DMA for _ in range(2 * NBUF)]
    if with_counts:
        scratch += [
            pltpu.VMEM((CS, 16), F32),
            pltpu.VMEM_SHARED((ACC_ROWS, 16), F32),
            pltpu.SemaphoreType.DMA,
        ]

    def body(y_hbm, src_hbm, dst_hbm, zero_hbm, *rest):
        if with_counts:
            (zero16_hbm, ones_hbm, out_hbm, cnt_hbm, src_v, dst_v) = rest[:6]
            rows = rest[6:6 + NBUF]
            acc = rest[6 + NBUF]
            gsem = rest[7 + NBUF:7 + 2 * NBUF]
            ssem = rest[7 + 2 * NBUF:7 + 3 * NBUF]
            ones_v, cacc, csem = rest[7 + 3 * NBUF:]
        else:
            (out_hbm, src_v, dst_v) = rest[:3]
            rows = rest[3:3 + NBUF]
            acc = rest[3 + NBUF]
            gsem = rest[4 + NBUF:4 + 2 * NBUF]
            ssem = rest[4 + 2 * NBUF:4 + 3 * NBUF]

        c = lax.axis_index("c")
        s = lax.axis_index("s")

        pltpu.sync_copy(src_hbm.at[s], src_v)
        pltpu.sync_copy(dst_hbm.at[s], dst_v)
        _zero_fill(zero_hbm, rows[0], acc, s * RPT, RPT, gsem[0], ssem[0])
        if with_counts:
            @pl.when(c == 0)
            def _():
                _zero_fill(zero16_hbm, ones_v, cacc, s * RPT, RPT,
                           gsem[1], ssem[1])
                pltpu.sync_copy(ones_hbm, ones_v)
        plsc.subcore_barrier()

        def start_g(j, b):
            pltpu.async_copy(y_hbm.at[c].at[src_v.at[j]], rows[b], gsem[b])

        def wait_g(j, b):
            pltpu.make_async_copy(
                y_hbm.at[c].at[src_v.at[j]], rows[b], gsem[b]).wait()

        def start_s(j, b):
            pltpu.async_copy(rows[b], acc.at[dst_v.at[j]], ssem[b], add=True)

        def wait_s(j, b):
            pltpu.make_async_copy(rows[b], acc.at[dst_v.at[j]], ssem[b]).wait()

        if with_counts:
            def start_c(j):
                pltpu.async_copy(ones_v, cacc.at[dst_v.at[j]], csem, add=True)

            def wait_c(j):
                pltpu.make_async_copy(ones_v, cacc.at[dst_v.at[j]], csem).wait()

        lead = NBUF // 2
        for j in range(lead):
            start_g(j, j)

        def group(i, carry):
            g = NBUF * i
            for b in range(NBUF):
                j = g + b
                wait_g(j, b)
                start_s(j, b)
                bn = (b + lead) % NBUF
                if b < lead:
                    @pl.when(i > 0)
                    def _():
                        wait_s(j - lead, bn)

                    start_g(j + lead, bn)
                else:
                    wait_s(j - lead, bn)

                    @pl.when(i < NCHUNK // NBUF - 1)
                    def _():
                        start_g(j + lead, bn)
            if with_counts:
                @pl.when((c == 0) & (i > 0))
                def _():
                    for k in range(NBUF):
                        wait_c(g - NBUF + k)

                @pl.when(c == 0)
                def _():
                    for k in range(NBUF):
                        start_c(g + k)

            return carry

        lax.fori_loop(0, NCHUNK // NBUF, group, 0)
        for j in range(NCHUNK - lead, NCHUNK):
            wait_s(j, j % NBUF)
        if with_counts:
            @pl.when(c == 0)
            def _():
                for k in range(NBUF):
                    wait_c(NCHUNK - NBUF + k)
        plsc.subcore_barrier()

        pltpu.sync_copy(acc.at[pl.ds(s * RPT, RPT)],
                        out_hbm.at[c].at[pl.ds(s * RPT, RPT)])
        if with_counts:
            @pl.when(c == 0)
            def _():
                pltpu.sync_copy(cacc.at[pl.ds(s * RPT, RPT)],
                                cnt_hbm.at[pl.ds(s * RPT, RPT)])

    return pl.kernel(body, out_type=out_type, mesh=_MESH,
                     scratch_types=scratch,
                     compiler_params=_SC_PARAMS)


_segsum64_cnt = _make_segsum(64, True)
_segsum64 = _make_segsum(64, False)
_segsum32 = _make_segsum(32, False)


_PREC = lax.Precision.HIGHEST
BN = 1000


def _row_spec(*dims):
    n = len(dims)

    def index_map(i):
        return tuple(i if d == n - 2 else 0 for d in range(n))

    return pl.BlockSpec(dims, index_map)


def _full_spec(shape):
    nd = len(shape)
    return pl.BlockSpec(shape, lambda i: (0,) * nd)


def _dense0_body(x_ref, wl0_ref, wl1_ref, wr_ref, b_ref, y_ref, z_ref):
    x = x_ref[...]
    y_ref[0] = jnp.dot(x, wl0_ref[...], preferred_element_type=F32,
                       precision=_PREC)
    y_ref[1] = jnp.dot(x, wl1_ref[...], preferred_element_type=F32,
                       precision=_PREC)
    z_ref[...] = jnp.dot(x, wr_ref[...], preferred_element_type=F32,
                         precision=_PREC) + b_ref[...]


def _combine_body(p_ref, cp_ref, z_ref, wl0_ref, wl1_ref, wr_ref, b_ref,
                  y_ref, z2_ref):
    cnt = cp_ref[:, 0:1]
    recip = 1.0 / jnp.maximum(cnt, 1.0)
    agg = jnp.concatenate([p_ref[0], p_ref[1]], axis=-1)
    h = jnp.maximum(agg * recip + z_ref[...], 0.0)
    y_ref[0] = jnp.dot(h, wl0_ref[...], preferred_element_type=F32,
                       precision=_PREC)
    y_ref[1] = jnp.dot(h, wl1_ref[...], preferred_element_type=F32,
                       precision=_PREC)
    z2_ref[...] = jnp.dot(h, wr_ref[...], preferred_element_type=F32,
                          precision=_PREC) + b_ref[...]


def _final_body(p_ref, cp_ref, z_ref, o_ref):
    cnt = cp_ref[:, 0:1]
    recip = 1.0 / jnp.maximum(cnt, 1.0)
    agg = jnp.concatenate([p_ref[0], p_ref[1]], axis=-1)
    h = jnp.maximum(agg * recip + z_ref[...], 0.0)
    m = jnp.max(h, axis=-1, keepdims=True)
    lse = jnp.log(jnp.sum(jnp.exp(h - m), axis=-1, keepdims=True)) + m
    o_ref[...] = h - lse


def _dense0(x, wl, wr, b):
    wh = wl.shape[1] // 2
    wo = wr.shape[1]
    return pl.pallas_call(
        _dense0_body,
        grid=(N // BN,),
        in_specs=[_row_spec(BN, 128),
                  _full_spec((128, wh)), _full_spec((128, wh)),
                  _full_spec((128, wo)), _full_spec((1, wo))],
        out_specs=[_row_spec(2, BN, wh), _row_spec(BN, wo)],
        out_shape=[jax.ShapeDtypeStruct((2, N, wh), F32),
                   jax.ShapeDtypeStruct((N, wo), F32)],
    )(x, wl[:, :wh], wl[:, wh:], wr, b.reshape(1, -1))


def _combine(p, cp, z, wl, wr, b):
    win = wl.shape[0]
    wh = wl.shape[1] // 2
    wo = wr.shape[1]
    return pl.pallas_call(
        _combine_body,
        grid=(N // BN,),
        in_specs=[_row_spec(2, BN, win // 2), _row_spec(BN, 16),
                  _row_spec(BN, win),
                  _full_spec((win, wh)), _full_spec((win, wh)),
                  _full_spec((win, wo)), _full_spec((1, wo))],
        out_specs=[_row_spec(2, BN, wh), _row_spec(BN, wo)],
        out_shape=[jax.ShapeDtypeStruct((2, N, wh), F32),
                   jax.ShapeDtypeStruct((N, wo), F32)],
    )(p, cp, z, wl[:, :wh], wl[:, wh:], wr, b.reshape(1, -1))


def _final(p, cp, z):
    wo = z.shape[1]
    return pl.pallas_call(
        _final_body,
        grid=(N // BN,),
        in_specs=[_row_spec(2, BN, wo // 2), _row_spec(BN, 16),
                  _row_spec(BN, wo)],
        out_specs=_row_spec(BN, wo),
        out_shape=jax.ShapeDtypeStruct((N, wo), F32),
    )(p, cp, z)


def kernel(x, edge_index, W1l, W1r, b1, W2l, W2r, b2, Wnl, Wnr, bn):
    src = edge_index[0]
    dst = edge_index[1]
    pad = EPAD - E
    src3 = jnp.concatenate(
        [src, jnp.zeros((pad,), jnp.int32)]).reshape(NS, NCHUNK, CS)
    dst3 = jnp.concatenate(
        [dst, jnp.full((pad,), N, jnp.int32)]).reshape(NS, NCHUNK, CS)

    z64 = jnp.zeros((8, 64), F32)
    z32 = jnp.zeros((8, 32), F32)
    z16 = jnp.zeros((8, 16), F32)
    ones = jnp.ones((CS, 16), F32)

    y1, z1 = _dense0(x, W1l, W1r, b1)
    p1, cp = _segsum64_cnt(y1, src3, dst3, z64, z16, ones)
    y2, z2 = _combine(p1, cp, z1, W2l, W2r, b2)
    p2 = _segsum64(y2, src3, dst3, z64)
    y3, z3 = _combine(p2, cp, z2, Wnl, Wnr, bn)
    p3 = _segsum32(y3, src3, dst3, z32)
    return _final(p3, cp, z3)


# speedup vs baseline: 1.0420x; 1.0420x over previous
import jax
import jax.numpy as jnp
from jax import lax
from jax.experimental import pallas as pl
from jax.experimental.pallas import tpu as pltpu
from jax.experimental.pallas import tpu_sc as plsc

N = 10000
E = 320000
NC = 2
NS = 16
CS = 128
NCHUNK = 160
EPT = CS * NCHUNK
EPAD = NS * EPT
ACC_ROWS = 10112
RPT = ACC_ROWS // NS
NBUF = 4
F32 = jnp.float32

_SC_PARAMS = pltpu.CompilerParams(use_tc_tiling_on_sc=False)
_MESH = plsc.VectorSubcoreMesh(
    core_axis_name="c", subcore_axis_name="s",
    num_cores=NC, num_subcores=NS)


def _make_segsum(WH, with_counts):
    if with_counts:
        out_type = [jax.ShapeDtypeStruct((NC, ACC_ROWS, WH), F32),
                    jax.ShapeDtypeStruct((ACC_ROWS, 16), F32)]
    else:
        out_type = jax.ShapeDtypeStruct((NC, ACC_ROWS, WH), F32)

    scratch = [
        pltpu.VMEM((NCHUNK, CS), jnp.int32),
        pltpu.VMEM((NCHUNK, CS), jnp.int32),
    ]
    scratch += [pltpu.VMEM((CS, WH), F32) for _ in range(NBUF)]
    scratch += [pltpu.VMEM_SHARED((ACC_ROWS, WH), F32)]
    scratch += [pltpu.SemaphoreType.DMA for _ in range(2 * NBUF)]
    if with_counts:
        scratch += [
            pltpu.VMEM((CS, 16), F32),
            pltpu.VMEM_SHARED((ACC_ROWS, 16), F32),
            pltpu.SemaphoreType.DMA,
        ]

    def body(y_hbm, src_hbm, dst_hbm, zero_hbm, *rest):
        if with_counts:
            (zero16_hbm, ones_hbm, out_hbm, cnt_hbm, src_v, dst_v) = rest[:6]
            rows = rest[6:6 + NBUF]
            acc = rest[6 + NBUF]
            gsem = rest[7 + NBUF:7 + 2 * NBUF]
            ssem = rest[7 + 2 * NBUF:7 + 3 * NBUF]
            ones_v, cacc, csem = rest[7 + 3 * NBUF:]
        else:
            (out_hbm, src_v, dst_v) = rest[:3]
            rows = rest[3:3 + NBUF]
            acc = rest[3 + NBUF]
            gsem = rest[4 + NBUF:4 + 2 * NBUF]
            ssem = rest[4 + 2 * NBUF:4 + 3 * NBUF]

        c = lax.axis_index("c")
        s = lax.axis_index("s")

        pltpu.sync_copy(src_hbm.at[s], src_v)
        pltpu.sync_copy(dst_hbm.at[s], dst_v)
        pltpu.sync_copy(zero_hbm, acc.at[pl.ds(s * RPT, RPT)])
        if with_counts:
            @pl.when(c == 0)
            def _():
                pltpu.sync_copy(zero16_hbm, cacc.at[pl.ds(s * RPT, RPT)])
                pltpu.sync_copy(ones_hbm, ones_v)
        plsc.subcore_barrier()

        def start_g(j, b):
            pltpu.async_copy(y_hbm.at[c].at[src_v.at[j]], rows[b], gsem[b])

        def wait_g(j, b):
            pltpu.make_async_copy(
                y_hbm.at[c].at[src_v.at[j]], rows[b], gsem[b]).wait()

        def start_s(j, b):
            pltpu.async_copy(rows[b], acc.at[dst_v.at[j]], ssem[b], add=True)

        def wait_s(j, b):
            pltpu.make_async_copy(rows[b], acc.at[dst_v.at[j]], ssem[b]).wait()

        if with_counts:
            def start_c(j):
                pltpu.async_copy(ones_v, cacc.at[dst_v.at[j]], csem, add=True)

            def wait_c(j):
                pltpu.make_async_copy(ones_v, cacc.at[dst_v.at[j]], csem).wait()

        lead = NBUF // 2
        for j in range(lead):
            start_g(j, j)

        def group(i, carry):
            g = NBUF * i
            for b in range(NBUF):
                j = g + b
                wait_g(j, b)
                start_s(j, b)
                bn = (b + lead) % NBUF
                if b < lead:
                    @pl.when(i > 0)
                    def _():
                        wait_s(j - lead, bn)

                    start_g(j + lead, bn)
                else:
                    wait_s(j - lead, bn)

                    @pl.when(i < NCHUNK // NBUF - 1)
                    def _():
                        start_g(j + lead, bn)
            if with_counts:
                @pl.when((c == 0) & (i > 0))
                def _():
                    for k in range(NBUF):
                        wait_c(g - NBUF + k)

                @pl.when(c == 0)
                def _():
                    for k in range(NBUF):
                        start_c(g + k)

            return carry

        lax.fori_loop(0, NCHUNK // NBUF, group, 0)
        for j in range(NCHUNK - lead, NCHUNK):
            wait_s(j, j % NBUF)
        if with_counts:
            @pl.when(c == 0)
            def _():
                for k in range(NBUF):
                    wait_c(NCHUNK - NBUF + k)
        plsc.subcore_barrier()

        pltpu.sync_copy(acc.at[pl.ds(s * RPT, RPT)],
                        out_hbm.at[c].at[pl.ds(s * RPT, RPT)])
        if with_counts:
            @pl.when(c == 0)
            def _():
                pltpu.sync_copy(cacc.at[pl.ds(s * RPT, RPT)],
                                cnt_hbm.at[pl.ds(s * RPT, RPT)])

    return pl.kernel(body, out_type=out_type, mesh=_MESH,
                     scratch_types=scratch,
                     compiler_params=_SC_PARAMS)


_segsum64_cnt = _make_segsum(64, True)
_segsum64 = _make_segsum(64, False)
_segsum32 = _make_segsum(32, False)


_PREC = lax.Precision.HIGHEST
BN = 1000


def _row_spec(*dims):
    n = len(dims)

    def index_map(i):
        return tuple(i if d == n - 2 else 0 for d in range(n))

    return pl.BlockSpec(dims, index_map)


def _full_spec(shape):
    nd = len(shape)
    return pl.BlockSpec(shape, lambda i: (0,) * nd)


def _dense0_body(x_ref, wl0_ref, wl1_ref, wr_ref, b_ref, y_ref, z_ref):
    x = x_ref[...]
    y_ref[0] = jnp.dot(x, wl0_ref[...], preferred_element_type=F32,
                       precision=_PREC)
    y_ref[1] = jnp.dot(x, wl1_ref[...], preferred_element_type=F32,
                       precision=_PREC)
    z_ref[...] = jnp.dot(x, wr_ref[...], preferred_element_type=F32,
                         precision=_PREC) + b_ref[...]


def _combine_body(p_ref, cp_ref, z_ref, wl0_ref, wl1_ref, wr_ref, b_ref,
                  y_ref, z2_ref):
    cnt = cp_ref[:, 0:1]
    recip = 1.0 / jnp.maximum(cnt, 1.0)
    agg = jnp.concatenate([p_ref[0], p_ref[1]], axis=-1)
    h = jnp.maximum(agg * recip + z_ref[...], 0.0)
    y_ref[0] = jnp.dot(h, wl0_ref[...], preferred_element_type=F32,
                       precision=_PREC)
    y_ref[1] = jnp.dot(h, wl1_ref[...], preferred_element_type=F32,
                       precision=_PREC)
    z2_ref[...] = jnp.dot(h, wr_ref[...], preferred_element_type=F32,
                          precision=_PREC) + b_ref[...]


def _final_body(p_ref, cp_ref, z_ref, o_ref):
    cnt = cp_ref[:, 0:1]
    recip = 1.0 / jnp.maximum(cnt, 1.0)
    agg = jnp.concatenate([p_ref[0], p_ref[1]], axis=-1)
    h = jnp.maximum(agg * recip + z_ref[...], 0.0)
    m = jnp.max(h, axis=-1, keepdims=True)
    lse = jnp.log(jnp.sum(jnp.exp(h - m), axis=-1, keepdims=True)) + m
    o_ref[...] = h - lse


def _dense0(x, wl, wr, b):
    wh = wl.shape[1] // 2
    wo = wr.shape[1]
    return pl.pallas_call(
        _dense0_body,
        grid=(N // BN,),
        in_specs=[_row_spec(BN, 128),
                  _full_spec((128, wh)), _full_spec((128, wh)),
                  _full_spec((128, wo)), _full_spec((1, wo))],
        out_specs=[_row_spec(2, BN, wh), _row_spec(BN, wo)],
        out_shape=[jax.ShapeDtypeStruct((2, N, wh), F32),
                   jax.ShapeDtypeStruct((N, wo), F32)],
    )(x, wl[:, :wh], wl[:, wh:], wr, b.reshape(1, -1))


def _combine(p, cp, z, wl, wr, b):
    win = wl.shape[0]
    wh = wl.shape[1] // 2
    wo = wr.shape[1]
    return pl.pallas_call(
        _combine_body,
        grid=(N // BN,),
        in_specs=[_row_spec(2, BN, win // 2), _row_spec(BN, 16),
                  _row_spec(BN, win),
                  _full_spec((win, wh)), _full_spec((win, wh)),
                  _full_spec((win, wo)), _full_spec((1, wo))],
        out_specs=[_row_spec(2, BN, wh), _row_spec(BN, wo)],
        out_shape=[jax.ShapeDtypeStruct((2, N, wh), F32),
                   jax.ShapeDtypeStruct((N, wo), F32)],
    )(p, cp, z, wl[:, :wh], wl[:, wh:], wr, b.reshape(1, -1))


def _final(p, cp, z):
    wo = z.shape[1]
    return pl.pallas_call(
        _final_body,
        grid=(N // BN,),
        in_specs=[_row_spec(2, BN, wo // 2), _row_spec(BN, 16),
                  _row_spec(BN, wo)],
        out_specs=_row_spec(BN, wo),
        out_shape=jax.ShapeDtypeStruct((N, wo), F32),
    )(p, cp, z)


def kernel(x, edge_index, W1l, W1r, b1, W2l, W2r, b2, Wnl, Wnr, bn):
    src = edge_index[0]
    dst = edge_index[1]
    pad = EPAD - E
    src3 = jnp.concatenate(
        [src, jnp.zeros((pad,), jnp.int32)]).reshape(NS, NCHUNK, CS)
    dst3 = jnp.concatenate(
        [dst, jnp.full((pad,), N, jnp.int32)]).reshape(NS, NCHUNK, CS)

    z64 = jnp.zeros((RPT, 64), F32)
    z32 = jnp.zeros((RPT, 32), F32)
    z16 = jnp.zeros((RPT, 16), F32)
    ones = jnp.ones((CS, 16), F32)

    y1, z1 = _dense0(x, W1l, W1r, b1)
    p1, cp = _segsum64_cnt(y1, src3, dst3, z64, z16, ones)
    y2, z2 = _combine(p1, cp, z1, W2l, W2r, b2)
    p2 = _segsum64(y2, src3, dst3, z64)
    y3, z3 = _combine(p2, cp, z2, Wnl, Wnr, bn)
    p3 = _segsum32(y3, src3, dst3, z32)
    return _final(p3, cp, z3)


# speedup vs baseline: 1.0866x; 1.0428x over previous
import jax
import jax.numpy as jnp
from jax import lax
from jax.experimental import pallas as pl
from jax.experimental.pallas import tpu as pltpu
from jax.experimental.pallas import tpu_sc as plsc

N = 10000
E = 320000
NC = 2
NS = 16
CS = 128
NCHUNK = 160
EPT = CS * NCHUNK
EPAD = NS * EPT
ACC_ROWS = 10112
RPT = ACC_ROWS // NS
NBUF = 4
F32 = jnp.float32

_SC_PARAMS = pltpu.CompilerParams(use_tc_tiling_on_sc=False)
_MESH = plsc.VectorSubcoreMesh(
    core_axis_name="c", subcore_axis_name="s",
    num_cores=NC, num_subcores=NS)


def _make_segsum(WH, with_counts):
    if with_counts:
        out_type = [jax.ShapeDtypeStruct((NC, ACC_ROWS, WH), F32),
                    jax.ShapeDtypeStruct((ACC_ROWS, 16), F32)]
    else:
        out_type = jax.ShapeDtypeStruct((NC, ACC_ROWS, WH), F32)

    scratch = [
        pltpu.VMEM((NCHUNK, CS), jnp.int32),
        pltpu.VMEM((NCHUNK, CS), jnp.int32),
    ]
    scratch += [pltpu.VMEM((CS, WH), F32) for _ in range(NBUF)]
    scratch += [pltpu.VMEM_SHARED((ACC_ROWS, WH), F32)]
    scratch += [pltpu.SemaphoreType.DMA for _ in range(2 * NBUF)]
    if with_counts:
        scratch += [
            pltpu.VMEM((CS, 16), F32),
            pltpu.VMEM_SHARED((ACC_ROWS, 16), F32),
            pltpu.SemaphoreType.DMA,
        ]

    def body(y_hbm, src_hbm, dst_hbm, zero_hbm, *rest):
        if with_counts:
            (zero16_hbm, ones_hbm, out_hbm, cnt_hbm, src_v, dst_v) = rest[:6]
            rows = rest[6:6 + NBUF]
            acc = rest[6 + NBUF]
            gsem = rest[7 + NBUF:7 + 2 * NBUF]
            ssem = rest[7 + 2 * NBUF:7 + 3 * NBUF]
            ones_v, cacc, csem = rest[7 + 3 * NBUF:]
        else:
            (out_hbm, src_v, dst_v) = rest[:3]
            rows = rest[3:3 + NBUF]
            acc = rest[3 + NBUF]
            gsem = rest[4 + NBUF:4 + 2 * NBUF]
            ssem = rest[4 + 2 * NBUF:4 + 3 * NBUF]

        c = lax.axis_index("c")
        s = lax.axis_index("s")

        pltpu.sync_copy(src_hbm.at[s], src_v)
        pltpu.sync_copy(dst_hbm.at[s], dst_v)
        pltpu.sync_copy(zero_hbm, acc.at[pl.ds(s * RPT, RPT)])
        if with_counts:
            @pl.when(c == 0)
            def _():
                pltpu.sync_copy(zero16_hbm, cacc.at[pl.ds(s * RPT, RPT)])
                pltpu.sync_copy(ones_hbm, ones_v)
        plsc.subcore_barrier()

        def start_g(j, b):
            pltpu.async_copy(y_hbm.at[c].at[src_v.at[j]], rows[b], gsem[b])

        def wait_g(j, b):
            pltpu.make_async_copy(
                y_hbm.at[c].at[src_v.at[j]], rows[b], gsem[b]).wait()

        def start_s(j, b):
            pltpu.async_copy(rows[b], acc.at[dst_v.at[j]], ssem[b], add=True)

        def wait_s(j, b):
            pltpu.make_async_copy(rows[b], acc.at[dst_v.at[j]], ssem[b]).wait()

        if with_counts:
            def start_c(j):
                pltpu.async_copy(ones_v, cacc.at[dst_v.at[j]], csem, add=True)

            def wait_c(j):
                pltpu.make_async_copy(ones_v, cacc.at[dst_v.at[j]], csem).wait()

        lead = NBUF // 2
        for j in range(lead):
            start_g(j, j)

        def group(i, carry):
            g = NBUF * i
            for b in range(NBUF):
                j = g + b
                wait_g(j, b)
                start_s(j, b)
                bn = (b + lead) % NBUF
                if b < lead:
                    @pl.when(i > 0)
                    def _():
                        wait_s(j - lead, bn)

                    start_g(j + lead, bn)
                else:
                    wait_s(j - lead, bn)

                    @pl.when(i < NCHUNK // NBUF - 1)
                    def _():
                        start_g(j + lead, bn)
            if with_counts:
                @pl.when((c == 0) & (i > 0))
                def _():
                    for k in range(NBUF):
                        wait_c(g - NBUF + k)

                @pl.when(c == 0)
                def _():
                    for k in range(NBUF):
                        start_c(g + k)

            return carry

        lax.fori_loop(0, NCHUNK // NBUF, group, 0)
        for j in range(NCHUNK - lead, NCHUNK):
            wait_s(j, j % NBUF)
        if with_counts:
            @pl.when(c == 0)
            def _():
                for k in range(NBUF):
                    wait_c(NCHUNK - NBUF + k)
        plsc.subcore_barrier()

        pltpu.sync_copy(acc.at[pl.ds(s * RPT, RPT)],
                        out_hbm.at[c].at[pl.ds(s * RPT, RPT)])
        if with_counts:
            @pl.when(c == 0)
            def _():
                pltpu.sync_copy(cacc.at[pl.ds(s * RPT, RPT)],
                                cnt_hbm.at[pl.ds(s * RPT, RPT)])

    return pl.kernel(body, out_type=out_type, mesh=_MESH,
                     scratch_types=scratch,
                     compiler_params=_SC_PARAMS)


_segsum64_cnt = _make_segsum(64, True)
_segsum64 = _make_segsum(64, False)
_segsum32 = _make_segsum(32, False)


_PREC = lax.Precision.DEFAULT
BN = 1000


def _row_spec(*dims):
    n = len(dims)

    def index_map(i):
        return tuple(i if d == n - 2 else 0 for d in range(n))

    return pl.BlockSpec(dims, index_map)


def _full_spec(shape):
    nd = len(shape)
    return pl.BlockSpec(shape, lambda i: (0,) * nd)


def _dense0_body(x_ref, wl0_ref, wl1_ref, wr_ref, b_ref, y_ref, z_ref):
    x = x_ref[...]
    y_ref[0] = jnp.dot(x, wl0_ref[...], preferred_element_type=F32,
                       precision=_PREC)
    y_ref[1] = jnp.dot(x, wl1_ref[...], preferred_element_type=F32,
                       precision=_PREC)
    z_ref[...] = jnp.dot(x, wr_ref[...], preferred_element_type=F32,
                         precision=_PREC) + b_ref[...]


def _combine_body(p_ref, cp_ref, z_ref, wl0_ref, wl1_ref, wr_ref, b_ref,
                  y_ref, z2_ref):
    cnt = cp_ref[:, 0:1]
    recip = 1.0 / jnp.maximum(cnt, 1.0)
    agg = jnp.concatenate([p_ref[0], p_ref[1]], axis=-1)
    h = jnp.maximum(agg * recip + z_ref[...], 0.0)
    y_ref[0] = jnp.dot(h, wl0_ref[...], preferred_element_type=F32,
                       precision=_PREC)
    y_ref[1] = jnp.dot(h, wl1_ref[...], preferred_element_type=F32,
                       precision=_PREC)
    z2_ref[...] = jnp.dot(h, wr_ref[...], preferred_element_type=F32,
                          precision=_PREC) + b_ref[...]


def _final_body(p_ref, cp_ref, z_ref, o_ref):
    cnt = cp_ref[:, 0:1]
    recip = 1.0 / jnp.maximum(cnt, 1.0)
    agg = jnp.concatenate([p_ref[0], p_ref[1]], axis=-1)
    h = jnp.maximum(agg * recip + z_ref[...], 0.0)
    m = jnp.max(h, axis=-1, keepdims=True)
    lse = jnp.log(jnp.sum(jnp.exp(h - m), axis=-1, keepdims=True)) + m
    o_ref[...] = h - lse


def _dense0(x, wl, wr, b):
    wh = wl.shape[1] // 2
    wo = wr.shape[1]
    return pl.pallas_call(
        _dense0_body,
        grid=(N // BN,),
        in_specs=[_row_spec(BN, 128),
                  _full_spec((128, wh)), _full_spec((128, wh)),
                  _full_spec((128, wo)), _full_spec((1, wo))],
        out_specs=[_row_spec(2, BN, wh), _row_spec(BN, wo)],
        out_shape=[jax.ShapeDtypeStruct((2, N, wh), F32),
                   jax.ShapeDtypeStruct((N, wo), F32)],
    )(x, wl[:, :wh], wl[:, wh:], wr, b.reshape(1, -1))


def _combine(p, cp, z, wl, wr, b):
    win = wl.shape[0]
    wh = wl.shape[1] // 2
    wo = wr.shape[1]
    return pl.pallas_call(
        _combine_body,
        grid=(N // BN,),
        in_specs=[_row_spec(2, BN, win // 2), _row_spec(BN, 16),
                  _row_spec(BN, win),
                  _full_spec((win, wh)), _full_spec((win, wh)),
                  _full_spec((win, wo)), _full_spec((1, wo))],
        out_specs=[_row_spec(2, BN, wh), _row_spec(BN, wo)],
        out_shape=[jax.ShapeDtypeStruct((2, N, wh), F32),
                   jax.ShapeDtypeStruct((N, wo), F32)],
    )(p, cp, z, wl[:, :wh], wl[:, wh:], wr, b.reshape(1, -1))


def _final(p, cp, z):
    wo = z.shape[1]
    return pl.pallas_call(
        _final_body,
        grid=(N // BN,),
        in_specs=[_row_spec(2, BN, wo // 2), _row_spec(BN, 16),
                  _row_spec(BN, wo)],
        out_specs=_row_spec(BN, wo),
        out_shape=jax.ShapeDtypeStruct((N, wo), F32),
    )(p, cp, z)


def kernel(x, edge_index, W1l, W1r, b1, W2l, W2r, b2, Wnl, Wnr, bn):
    src = edge_index[0]
    dst = edge_index[1]
    pad = EPAD - E
    src3 = jnp.concatenate(
        [src, jnp.zeros((pad,), jnp.int32)]).reshape(NS, NCHUNK, CS)
    dst3 = jnp.concatenate(
        [dst, jnp.full((pad,), N, jnp.int32)]).reshape(NS, NCHUNK, CS)

    z64 = jnp.zeros((RPT, 64), F32)
    z32 = jnp.zeros((RPT, 32), F32)
    z16 = jnp.zeros((RPT, 16), F32)
    ones = jnp.ones((CS, 16), F32)

    y1, z1 = _dense0(x, W1l, W1r, b1)
    p1, cp = _segsum64_cnt(y1, src3, dst3, z64, z16, ones)
    y2, z2 = _combine(p1, cp, z1, W2l, W2r, b2)
    p2 = _segsum64(y2, src3, dst3, z64)
    y3, z3 = _combine(p2, cp, z2, Wnl, Wnr, bn)
    p3 = _segsum32(y3, src3, dst3, z32)
    return _final(p3, cp, z3)


# speedup vs baseline: 1.0975x; 1.0100x over previous
import jax
import jax.numpy as jnp
from jax import lax
from jax.experimental import pallas as pl
from jax.experimental.pallas import tpu as pltpu
from jax.experimental.pallas import tpu_sc as plsc

N = 10000
E = 320000
NC = 2
NS = 16
CS = 128
NCHUNK = 160
EPT = CS * NCHUNK
EPAD = NS * EPT
ACC_ROWS = 10112
RPT = ACC_ROWS // NS
NBUF = 4
F32 = jnp.float32

_SC_PARAMS = pltpu.CompilerParams(use_tc_tiling_on_sc=False)
_MESH = plsc.VectorSubcoreMesh(
    core_axis_name="c", subcore_axis_name="s",
    num_cores=NC, num_subcores=NS)


def _make_segsum(WH, with_counts):
    if with_counts:
        out_type = [jax.ShapeDtypeStruct((NC, ACC_ROWS, WH), F32),
                    jax.ShapeDtypeStruct((ACC_ROWS, 16), F32)]
    else:
        out_type = jax.ShapeDtypeStruct((NC, ACC_ROWS, WH), F32)

    scratch = [
        pltpu.VMEM((NCHUNK, CS), jnp.int32),
        pltpu.VMEM((NCHUNK, CS), jnp.int32),
    ]
    scratch += [pltpu.VMEM((CS, WH), F32) for _ in range(NBUF)]
    scratch += [pltpu.VMEM_SHARED((ACC_ROWS, WH), F32)]
    scratch += [pltpu.SemaphoreType.DMA for _ in range(2 * NBUF)]
    if with_counts:
        scratch += [
            pltpu.VMEM((CS, 16), F32),
            pltpu.VMEM_SHARED((ACC_ROWS, 16), F32),
            pltpu.SemaphoreType.DMA,
        ]

    def body(y_hbm, src_hbm, dst_hbm, zero_hbm, *rest):
        if with_counts:
            (zero16_hbm, ones_hbm, out_hbm, cnt_hbm, src_v, dst_v) = rest[:6]
            rows = rest[6:6 + NBUF]
            acc = rest[6 + NBUF]
            gsem = rest[7 + NBUF:7 + 2 * NBUF]
            ssem = rest[7 + 2 * NBUF:7 + 3 * NBUF]
            ones_v, cacc, csem = rest[7 + 3 * NBUF:]
        else:
            (out_hbm, src_v, dst_v) = rest[:3]
            rows = rest[3:3 + NBUF]
            acc = rest[3 + NBUF]
            gsem = rest[4 + NBUF:4 + 2 * NBUF]
            ssem = rest[4 + 2 * NBUF:4 + 3 * NBUF]

        c = lax.axis_index("c")
        s = lax.axis_index("s")

        pltpu.sync_copy(src_hbm.at[s], src_v)
        pltpu.sync_copy(dst_hbm.at[s], dst_v)
        pltpu.sync_copy(zero_hbm, acc.at[pl.ds(s * RPT, RPT)])
        if with_counts:
            @pl.when(c == 0)
            def _():
                pltpu.sync_copy(zero16_hbm, cacc.at[pl.ds(s * RPT, RPT)])
                pltpu.sync_copy(ones_hbm, ones_v)
        plsc.subcore_barrier()

        def start_g(j, b):
            pltpu.async_copy(y_hbm.at[c].at[src_v.at[j]], rows[b], gsem[b])

        def wait_g(j, b):
            pltpu.make_async_copy(
                y_hbm.at[c].at[src_v.at[j]], rows[b], gsem[b]).wait()

        def start_s(j, b):
            pltpu.async_copy(rows[b], acc.at[dst_v.at[j]], ssem[b], add=True)

        def wait_s(j, b):
            pltpu.make_async_copy(rows[b], acc.at[dst_v.at[j]], ssem[b]).wait()

        if with_counts:
            def start_c(j):
                pltpu.async_copy(ones_v, cacc.at[dst_v.at[j]], csem, add=True)

            def wait_c(j):
                pltpu.make_async_copy(ones_v, cacc.at[dst_v.at[j]], csem).wait()

        lead = NBUF // 2
        for j in range(lead):
            start_g(j, j)

        def group(i, carry):
            g = NBUF * i
            for b in range(NBUF):
                j = g + b
                wait_g(j, b)
                start_s(j, b)
                bn = (b + lead) % NBUF
                if b < lead:
                    @pl.when(i > 0)
                    def _():
                        wait_s(j - lead, bn)

                    start_g(j + lead, bn)
                else:
                    wait_s(j - lead, bn)

                    @pl.when(i < NCHUNK // NBUF - 1)
                    def _():
                        start_g(j + lead, bn)
            if with_counts:
                @pl.when((c == 0) & (i > 0))
                def _():
                    for k in range(NBUF):
                        wait_c(g - NBUF + k)

                @pl.when(c == 0)
                def _():
                    for k in range(NBUF):
                        start_c(g + k)

            return carry

        lax.fori_loop(0, NCHUNK // NBUF, group, 0)
        for j in range(NCHUNK - lead, NCHUNK):
            wait_s(j, j % NBUF)
        if with_counts:
            @pl.when(c == 0)
            def _():
                for k in range(NBUF):
                    wait_c(NCHUNK - NBUF + k)
        plsc.subcore_barrier()

        pltpu.sync_copy(acc.at[pl.ds(s * RPT, RPT)],
                        out_hbm.at[c].at[pl.ds(s * RPT, RPT)])
        if with_counts:
            @pl.when(c == 0)
            def _():
                pltpu.sync_copy(cacc.at[pl.ds(s * RPT, RPT)],
                                cnt_hbm.at[pl.ds(s * RPT, RPT)])

    return pl.kernel(body, out_type=out_type, mesh=_MESH,
                     scratch_types=scratch,
                     compiler_params=_SC_PARAMS)


_segsum64_cnt = _make_segsum(64, True)
_segsum64 = _make_segsum(64, False)
_segsum32 = _make_segsum(32, False)


_PREC = lax.Precision.DEFAULT
BN = 2000


def _row_spec(*dims):
    n = len(dims)

    def index_map(i):
        return tuple(i if d == n - 2 else 0 for d in range(n))

    return pl.BlockSpec(dims, index_map)


def _full_spec(shape):
    nd = len(shape)
    return pl.BlockSpec(shape, lambda i: (0,) * nd)


def _dense0_body(x_ref, wl0_ref, wl1_ref, wr_ref, b_ref, y_ref, z_ref):
    x = x_ref[...]
    y_ref[0] = jnp.dot(x, wl0_ref[...], preferred_element_type=F32,
                       precision=_PREC)
    y_ref[1] = jnp.dot(x, wl1_ref[...], preferred_element_type=F32,
                       precision=_PREC)
    z_ref[...] = jnp.dot(x, wr_ref[...], preferred_element_type=F32,
                         precision=_PREC) + b_ref[...]


def _combine_body(p_ref, cp_ref, z_ref, wl0_ref, wl1_ref, wr_ref, b_ref,
                  y_ref, z2_ref):
    cnt = cp_ref[:, 0:1]
    recip = 1.0 / jnp.maximum(cnt, 1.0)
    agg = jnp.concatenate([p_ref[0], p_ref[1]], axis=-1)
    h = jnp.maximum(agg * recip + z_ref[...], 0.0)
    y_ref[0] = jnp.dot(h, wl0_ref[...], preferred_element_type=F32,
                       precision=_PREC)
    y_ref[1] = jnp.dot(h, wl1_ref[...], preferred_element_type=F32,
                       precision=_PREC)
    z2_ref[...] = jnp.dot(h, wr_ref[...], preferred_element_type=F32,
                          precision=_PREC) + b_ref[...]


def _final_body(p_ref, cp_ref, z_ref, o_ref):
    cnt = cp_ref[:, 0:1]
    recip = 1.0 / jnp.maximum(cnt, 1.0)
    agg = jnp.concatenate([p_ref[0], p_ref[1]], axis=-1)
    h = jnp.maximum(agg * recip + z_ref[...], 0.0)
    m = jnp.max(h, axis=-1, keepdims=True)
    lse = jnp.log(jnp.sum(jnp.exp(h - m), axis=-1, keepdims=True)) + m
    o_ref[...] = h - lse


def _dense0(x, wl, wr, b):
    wh = wl.shape[1] // 2
    wo = wr.shape[1]
    return pl.pallas_call(
        _dense0_body,
        grid=(N // BN,),
        in_specs=[_row_spec(BN, 128),
                  _full_spec((128, wh)), _full_spec((128, wh)),
                  _full_spec((128, wo)), _full_spec((1, wo))],
        out_specs=[_row_spec(2, BN, wh), _row_spec(BN, wo)],
        out_shape=[jax.ShapeDtypeStruct((2, N, wh), F32),
                   jax.ShapeDtypeStruct((N, wo), F32)],
    )(x, wl[:, :wh], wl[:, wh:], wr, b.reshape(1, -1))


def _combine(p, cp, z, wl, wr, b):
    win = wl.shape[0]
    wh = wl.shape[1] // 2
    wo = wr.shape[1]
    return pl.pallas_call(
        _combine_body,
        grid=(N // BN,),
        in_specs=[_row_spec(2, BN, win // 2), _row_spec(BN, 16),
                  _row_spec(BN, win),
                  _full_spec((win, wh)), _full_spec((win, wh)),
                  _full_spec((win, wo)), _full_spec((1, wo))],
        out_specs=[_row_spec(2, BN, wh), _row_spec(BN, wo)],
        out_shape=[jax.ShapeDtypeStruct((2, N, wh), F32),
                   jax.ShapeDtypeStruct((N, wo), F32)],
    )(p, cp, z, wl[:, :wh], wl[:, wh:], wr, b.reshape(1, -1))


def _final(p, cp, z):
    wo = z.shape[1]
    return pl.pallas_call(
        _final_body,
        grid=(N // BN,),
        in_specs=[_row_spec(2, BN, wo // 2), _row_spec(BN, 16),
                  _row_spec(BN, wo)],
        out_specs=_row_spec(BN, wo),
        out_shape=jax.ShapeDtypeStruct((N, wo), F32),
    )(p, cp, z)


def kernel(x, edge_index, W1l, W1r, b1, W2l, W2r, b2, Wnl, Wnr, bn):
    src = edge_index[0]
    dst = edge_index[1]
    pad = EPAD - E
    src3 = jnp.concatenate(
        [src, jnp.zeros((pad,), jnp.int32)]).reshape(NS, NCHUNK, CS)
    dst3 = jnp.concatenate(
        [dst, jnp.full((pad,), N, jnp.int32)]).reshape(NS, NCHUNK, CS)

    z64 = jnp.zeros((RPT, 64), F32)
    z32 = jnp.zeros((RPT, 32), F32)
    z16 = jnp.zeros((RPT, 16), F32)
    ones = jnp.ones((CS, 16), F32)

    y1, z1 = _dense0(x, W1l, W1r, b1)
    p1, cp = _segsum64_cnt(y1, src3, dst3, z64, z16, ones)
    y2, z2 = _combine(p1, cp, z1, W2l, W2r, b2)
    p2 = _segsum64(y2, src3, dst3, z64)
    y3, z3 = _combine(p2, cp, z2, Wnl, Wnr, bn)
    p3 = _segsum32(y3, src3, dst3, z32)
    return _final(p3, cp, z3)
